# Initial kernel scaffold; baseline (speedup 1.0000x reference)
import jax
import jax.numpy as jnp
from jax.experimental import pallas as pl


def kernel(input_id, word_table, pos_table, gamma, beta):
    raise NotImplementedError("write your pallas kernel here")


# speedup vs baseline: 3.2338x; 3.2338x over previous
import functools

import jax
import jax.numpy as jnp
from jax import lax
from jax.experimental import pallas as pl
from jax.experimental.pallas import tpu as pltpu
from jax.experimental.pallas import tpu_sc as plsc

_HID = 64
_NREG = _HID // 16
_NC, _NS = 2, 16
_NW = _NC * _NS
_C = 512
_IG = 128
_G = _C // _IG


def _rsqrt16(x):
    i = lax.bitcast_convert_type(x, jnp.int32)
    y = lax.bitcast_convert_type(jnp.int32(0x5F3759DF) - (i >> 1), jnp.float32)
    hx = x * 0.5
    for _ in range(3):
        y = y * (1.5 - hx * y * y)
    return y


def _make_body(n_rows, seq_len, eps):
    rows_per_w = n_rows // _NW
    n_chunks = rows_per_w // _C

    def body(idx_hbm, word_hbm, pos_hbm, gamma_hbm, beta_hbm, out_hbm,
             idx_v, rows_v, pos_v, gamma_v, beta_v, sem):
        wid = lax.axis_index("s") * _NC + lax.axis_index("c")
        base = wid * rows_per_w
        pltpu.sync_copy(pos_hbm, pos_v)
        pltpu.sync_copy(gamma_hbm, gamma_v)
        pltpu.sync_copy(beta_hbm, beta_v)
        g_regs = [gamma_v[pl.ds(16 * k, 16)] for k in range(_NREG)]
        b_regs = [beta_v[pl.ds(16 * k, 16)] for k in range(_NREG)]

        def chunk_body(g, carry):
            cbase = base + g * _C
            pltpu.sync_copy(idx_hbm.at[pl.ds(lax.div(cbase, _IG), _G)], idx_v)
            cps = [
                pltpu.async_copy(word_hbm.at[idx_v.at[j]],
                                 rows_v.at[pl.ds(j * _IG, _IG)], sem)
                for j in range(_G)
            ]
            for c in cps:
                c.wait()

            lanes = lax.iota(jnp.int32, 16)
            perms = [lanes ^ d for d in (1, 2, 4, 8)]

            def row_body(r, rcarry):
                p = lax.rem(cbase + r, seq_len)
                s = [rows_v[r, pl.ds(16 * k, 16)] + pos_v[p, pl.ds(16 * k, 16)]
                     for k in range(_NREG)]
                t = (s[0] + s[1]) + (s[2] + s[3])
                u = (s[0] * s[0] + s[1] * s[1]) + (s[2] * s[2] + s[3] * s[3])
                for perm in perms:
                    t = t + t.at[perm].get(mode="promise_in_bounds")
                    u = u + u.at[perm].get(mode="promise_in_bounds")
                m = t * (1.0 / _HID)
                var = jnp.maximum(u * (1.0 / _HID) - m * m, 0.0) + eps
                rs = _rsqrt16(var)
                for k in range(_NREG):
                    y = (s[k] - m) * rs
                    rows_v[r, pl.ds(16 * k, 16)] = y * g_regs[k] + b_regs[k]
                return rcarry

            lax.fori_loop(0, _C, row_body, 0, unroll=2)
            pltpu.sync_copy(rows_v, out_hbm.at[pl.ds(cbase, _C)])
            return carry

        lax.fori_loop(0, n_chunks, chunk_body, 0)

    return body


def kernel(input_id, word_table, pos_table, gamma, beta):
    b, seq_len = input_id.shape
    n_rows = b * seq_len
    idx2d = input_id.reshape(n_rows // _IG, _IG).astype(jnp.int32)
    pos = pos_table[:seq_len]
    mesh = plsc.VectorSubcoreMesh(core_axis_name="c", subcore_axis_name="s")
    fn = functools.partial(
        pl.kernel,
        mesh=mesh,
        compiler_params=pltpu.CompilerParams(use_tc_tiling_on_sc=False),
        out_type=jax.ShapeDtypeStruct((n_rows, _HID), jnp.float32),
        scratch_types=[
            pltpu.VMEM((_G, _IG), jnp.int32),
            pltpu.VMEM((_C, _HID), jnp.float32),
            pltpu.VMEM((seq_len, _HID), jnp.float32),
            pltpu.VMEM((_HID,), jnp.float32),
            pltpu.VMEM((_HID,), jnp.float32),
            pltpu.SemaphoreType.DMA,
        ],
    )(_make_body(n_rows, seq_len, 1e-12))
    out = fn(idx2d, word_table, pos, gamma, beta)
    return out.reshape(b, seq_len, _HID)


# speedup vs baseline: 3.4424x; 1.0645x over previous
import functools

import jax
import jax.numpy as jnp
from jax import lax
from jax.experimental import pallas as pl
from jax.experimental.pallas import tpu as pltpu
from jax.experimental.pallas import tpu_sc as plsc

_HID = 64
_NREG = _HID // 16
_NC, _NS = 2, 16
_NW = _NC * _NS
_C = 512
_IG = 128
_G = _C // _IG


def _rsqrt16(x):
    i = lax.bitcast_convert_type(x, jnp.int32)
    y = lax.bitcast_convert_type(jnp.int32(0x5F3759DF) - (i >> 1), jnp.float32)
    hx = x * 0.5
    for _ in range(2):
        y = y * (1.5 - hx * y * y)
    return y


def _make_body(n_rows, seq_len, eps):
    rows_per_w = n_rows // _NW
    n_chunks = rows_per_w // _C

    def body(idx_hbm, word_hbm, pos_hbm, gamma_hbm, beta_hbm, out_hbm,
             idx_v, rows_v, pos_v, gamma_v, beta_v, sem):
        wid = lax.axis_index("s") * _NC + lax.axis_index("c")
        base = wid * rows_per_w
        pltpu.sync_copy(pos_hbm, pos_v)
        pltpu.sync_copy(gamma_hbm, gamma_v)
        pltpu.sync_copy(beta_hbm, beta_v)
        g_regs = [gamma_v[pl.ds(16 * k, 16)] for k in range(_NREG)]
        b_regs = [beta_v[pl.ds(16 * k, 16)] for k in range(_NREG)]

        def chunk_body(g, carry):
            cbase = base + g * _C
            pltpu.sync_copy(idx_hbm.at[pl.ds(lax.div(cbase, _IG), _G)], idx_v)
            cps = [
                pltpu.async_copy(word_hbm.at[idx_v.at[j]],
                                 rows_v.at[pl.ds(j * _IG, _IG)], sem)
                for j in range(_G)
            ]
            for c in cps:
                c.wait()

            lanes = lax.iota(jnp.int32, 16)
            perms = [lanes ^ d for d in (1, 2, 4, 8)]

            def row_body(r, p):
                s = [rows_v[r, pl.ds(16 * k, 16)] + pos_v[p, pl.ds(16 * k, 16)]
                     for k in range(_NREG)]
                t = (s[0] + s[1]) + (s[2] + s[3])
                u = (s[0] * s[0] + s[1] * s[1]) + (s[2] * s[2] + s[3] * s[3])
                for perm in perms:
                    t = t + t.at[perm].get(mode="promise_in_bounds")
                    u = u + u.at[perm].get(mode="promise_in_bounds")
                m = t * (1.0 / _HID)
                var = jnp.maximum(u * (1.0 / _HID) - m * m, 0.0) + eps
                rs = _rsqrt16(var)
                for k in range(_NREG):
                    y = (s[k] - m) * rs
                    rows_v[r, pl.ds(16 * k, 16)] = y * g_regs[k] + b_regs[k]
                return jnp.where(p == seq_len - 1, 0, p + 1)

            lax.fori_loop(0, _C, row_body, lax.rem(cbase, seq_len), unroll=4)
            pltpu.sync_copy(rows_v, out_hbm.at[pl.ds(cbase, _C)])
            return carry

        lax.fori_loop(0, n_chunks, chunk_body, 0)

    return body


def kernel(input_id, word_table, pos_table, gamma, beta):
    b, seq_len = input_id.shape
    n_rows = b * seq_len
    idx2d = input_id.reshape(n_rows // _IG, _IG).astype(jnp.int32)
    pos = pos_table[:seq_len]
    mesh = plsc.VectorSubcoreMesh(core_axis_name="c", subcore_axis_name="s")
    fn = functools.partial(
        pl.kernel,
        mesh=mesh,
        compiler_params=pltpu.CompilerParams(use_tc_tiling_on_sc=False),
        out_type=jax.ShapeDtypeStruct((n_rows, _HID), jnp.float32),
        scratch_types=[
            pltpu.VMEM((_G, _IG), jnp.int32),
            pltpu.VMEM((_C, _HID), jnp.float32),
            pltpu.VMEM((seq_len, _HID), jnp.float32),
            pltpu.VMEM((_HID,), jnp.float32),
            pltpu.VMEM((_HID,), jnp.float32),
            pltpu.SemaphoreType.DMA,
        ],
    )(_make_body(n_rows, seq_len, 1e-12))
    out = fn(idx2d, word_table, pos, gamma, beta)
    return out.reshape(b, seq_len, _HID)


# speedup vs baseline: 5.5962x; 1.6257x over previous
import functools

import jax
import jax.numpy as jnp
from jax import lax
from jax.experimental import pallas as pl
from jax.experimental.pallas import tpu as pltpu
from jax.experimental.pallas import tpu_sc as plsc

_HID = 64
_NREG = _HID // 16
_NC, _NS = 2, 16
_NW = _NC * _NS
_C = 512
_IG = 128
_G = _C // _IG


def _rsqrt16(x):
    i = lax.bitcast_convert_type(x, jnp.int32)
    y = lax.bitcast_convert_type(jnp.int32(0x5F3759DF) - (i >> 1), jnp.float32)
    hx = x * 0.5
    for _ in range(2):
        y = y * (1.5 - hx * y * y)
    return y


def _make_body(n_rows, seq_len, eps):
    rows_per_w = n_rows // _NW
    n_chunks = rows_per_w // _C

    def body(idx_hbm, word_hbm, pos_hbm, gamma_hbm, beta_hbm, out_hbm,
             idx_v, rows_v, pos_v, gamma_v, beta_v, sem):
        wid = lax.axis_index("s") * _NC + lax.axis_index("c")
        base = wid * rows_per_w
        pltpu.sync_copy(pos_hbm, pos_v)
        pltpu.sync_copy(gamma_hbm, gamma_v)
        pltpu.sync_copy(beta_hbm, beta_v)
        g_regs = [gamma_v[pl.ds(16 * k, 16)] for k in range(_NREG)]
        b_regs = [beta_v[pl.ds(16 * k, 16)] for k in range(_NREG)]

        def chunk_body(g, carry):
            cbase = base + g * _C
            pltpu.sync_copy(idx_hbm.at[pl.ds(lax.div(cbase, _IG), _G)], idx_v)
            cps = [
                pltpu.async_copy(word_hbm.at[idx_v.at[j]],
                                 rows_v.at[pl.ds(j * _IG, _IG)], sem)
                for j in range(_G)
            ]
            for c in cps:
                c.wait()

            lanes = lax.iota(jnp.int32, 16)
            perms = [lanes ^ d for d in (1, 2, 4, 8)]
            off = lax.rem(cbase, seq_len)

            @plsc.parallel_loop(0, _C, unroll=4)
            def row_body(r):
                p = r + off
                for _ in range((_C + seq_len - 1) // seq_len + 1):
                    p = jnp.where(p >= seq_len, p - seq_len, p)
                s = [rows_v[r, pl.ds(16 * k, 16)] + pos_v[p, pl.ds(16 * k, 16)]
                     for k in range(_NREG)]
                t = (s[0] + s[1]) + (s[2] + s[3])
                u = (s[0] * s[0] + s[1] * s[1]) + (s[2] * s[2] + s[3] * s[3])
                for perm in perms:
                    t = t + t.at[perm].get(mode="promise_in_bounds")
                    u = u + u.at[perm].get(mode="promise_in_bounds")
                m = t * (1.0 / _HID)
                var = jnp.maximum(u * (1.0 / _HID) - m * m, 0.0) + eps
                rs = _rsqrt16(var)
                for k in range(_NREG):
                    y = (s[k] - m) * rs
                    rows_v[r, pl.ds(16 * k, 16)] = y * g_regs[k] + b_regs[k]
            pltpu.sync_copy(rows_v, out_hbm.at[pl.ds(cbase, _C)])
            return carry

        lax.fori_loop(0, n_chunks, chunk_body, 0)

    return body


def kernel(input_id, word_table, pos_table, gamma, beta):
    b, seq_len = input_id.shape
    n_rows = b * seq_len
    idx2d = input_id.reshape(n_rows // _IG, _IG).astype(jnp.int32)
    pos = pos_table[:seq_len]
    mesh = plsc.VectorSubcoreMesh(core_axis_name="c", subcore_axis_name="s")
    fn = functools.partial(
        pl.kernel,
        mesh=mesh,
        compiler_params=pltpu.CompilerParams(use_tc_tiling_on_sc=False),
        out_type=jax.ShapeDtypeStruct((n_rows, _HID), jnp.float32),
        scratch_types=[
            pltpu.VMEM((_G, _IG), jnp.int32),
            pltpu.VMEM((_C, _HID), jnp.float32),
            pltpu.VMEM((seq_len, _HID), jnp.float32),
            pltpu.VMEM((_HID,), jnp.float32),
            pltpu.VMEM((_HID,), jnp.float32),
            pltpu.SemaphoreType.DMA,
        ],
    )(_make_body(n_rows, seq_len, 1e-12))
    out = fn(idx2d, word_table, pos, gamma, beta)
    return out.reshape(b, seq_len, _HID)


# speedup vs baseline: 6.3708x; 1.1384x over previous
import functools

import jax
import jax.numpy as jnp
from jax import lax
from jax.experimental import pallas as pl
from jax.experimental.pallas import tpu as pltpu
from jax.experimental.pallas import tpu_sc as plsc

_HID = 64
_NREG = _HID // 16
_NC, _NS = 2, 16
_NW = _NC * _NS
_C = 256
_IG = 128
_G = _C // _IG


def _rsqrt16(x):
    i = lax.bitcast_convert_type(x, jnp.int32)
    y = lax.bitcast_convert_type(jnp.int32(0x5F3759DF) - (i >> 1), jnp.float32)
    hx = x * 0.5
    for _ in range(2):
        y = y * (1.5 - hx * y * y)
    return y


def _make_body(n_rows, seq_len, eps):
    rows_per_w = n_rows // _NW
    n_chunks = rows_per_w // _C
    n_pairs = n_chunks // 2

    def body(idx_hbm, word_hbm, pos_hbm, gamma_hbm, beta_hbm, out_hbm,
             idx0, idx1, rows0, rows1, sb0, sb1, pos_v, gamma_v, beta_v,
             gsem0, gsem1, ssem0, ssem1):
        idx_b = (idx0, idx1)
        rows_b = (rows0, rows1)
        sb_b = (sb0, sb1)
        gsem_b = (gsem0, gsem1)
        ssem_b = (ssem0, ssem1)

        wid = lax.axis_index("s") * _NC + lax.axis_index("c")
        base = wid * rows_per_w
        pltpu.sync_copy(pos_hbm, pos_v)
        pltpu.sync_copy(gamma_hbm, gamma_v)
        pltpu.sync_copy(beta_hbm, beta_v)
        g_regs = [gamma_v[pl.ds(16 * k, 16)] for k in range(_NREG)]
        b_regs = [beta_v[pl.ds(16 * k, 16)] for k in range(_NREG)]
        lanes = lax.iota(jnp.int32, 16)
        perms = [lanes ^ d for d in (1, 2, 4, 8)]

        def fire_gather(buf, g):
            cbase = base + g * _C
            pltpu.sync_copy(idx_hbm.at[pl.ds(lax.div(cbase, _IG), _G)],
                            idx_b[buf])
            for j in range(_G):
                pltpu.async_copy(word_hbm.at[idx_b[buf].at[j]],
                                 rows_b[buf].at[pl.ds(j * _IG, _IG)],
                                 gsem_b[buf])

        def wait_gather(buf):
            pltpu.make_async_copy(word_hbm.at[pl.ds(0, _C)], rows_b[buf],
                                  gsem_b[buf]).wait()

        def wait_store(buf):
            pltpu.make_async_copy(sb_b[buf], out_hbm.at[pl.ds(0, _C)],
                                  ssem_b[buf]).wait()

        fire_gather(0, 0)

        def pair_body(tt, carry):
            for par in (0, 1):
                cur = 2 * tt + par
                cbase = base + cur * _C

                @pl.when(cur + 1 < n_chunks)
                def _():
                    fire_gather(1 - par, cur + 1)

                wait_gather(par)

                @pl.when(cur >= 2)
                def _():
                    wait_store(par)

                rows_v = rows_b[par]
                sb_v = sb_b[par]
                off = lax.rem(cbase, seq_len)

                @plsc.parallel_loop(0, _C, unroll=4)
                def row_body(r):
                    p = r + off
                    for _ in range((_C + seq_len - 1) // seq_len + 1):
                        p = jnp.where(p >= seq_len, p - seq_len, p)
                    s = [rows_v[r, pl.ds(16 * k, 16)]
                         + pos_v[p, pl.ds(16 * k, 16)] for k in range(_NREG)]
                    t = (s[0] + s[1]) + (s[2] + s[3])
                    u = (s[0] * s[0] + s[1] * s[1]) + (s[2] * s[2] + s[3] * s[3])
                    for perm in perms:
                        t = t + t.at[perm].get(mode="promise_in_bounds")
                        u = u + u.at[perm].get(mode="promise_in_bounds")
                    m = t * (1.0 / _HID)
                    var = jnp.maximum(u * (1.0 / _HID) - m * m, 0.0) + eps
                    rs = _rsqrt16(var)
                    for k in range(_NREG):
                        y = (s[k] - m) * rs
                        sb_v[r, pl.ds(16 * k, 16)] = y * g_regs[k] + b_regs[k]

                pltpu.async_copy(sb_v, out_hbm.at[pl.ds(cbase, _C)],
                                 ssem_b[par])
            return carry

        lax.fori_loop(0, n_pairs, pair_body, 0)
        wait_store(0)
        wait_store(1)

    return body


def kernel(input_id, word_table, pos_table, gamma, beta):
    b, seq_len = input_id.shape
    n_rows = b * seq_len
    idx2d = input_id.reshape(n_rows // _IG, _IG).astype(jnp.int32)
    pos = pos_table[:seq_len]
    mesh = plsc.VectorSubcoreMesh(core_axis_name="c", subcore_axis_name="s")
    fn = functools.partial(
        pl.kernel,
        mesh=mesh,
        compiler_params=pltpu.CompilerParams(use_tc_tiling_on_sc=False),
        out_type=jax.ShapeDtypeStruct((n_rows, _HID), jnp.float32),
        scratch_types=[
            pltpu.VMEM((_G, _IG), jnp.int32),
            pltpu.VMEM((_G, _IG), jnp.int32),
            pltpu.VMEM((_C, _HID), jnp.float32),
            pltpu.VMEM((_C, _HID), jnp.float32),
            pltpu.VMEM((_C, _HID), jnp.float32),
            pltpu.VMEM((_C, _HID), jnp.float32),
            pltpu.VMEM((seq_len, _HID), jnp.float32),
            pltpu.VMEM((_HID,), jnp.float32),
            pltpu.VMEM((_HID,), jnp.float32),
            pltpu.SemaphoreType.DMA,
            pltpu.SemaphoreType.DMA,
            pltpu.SemaphoreType.DMA,
            pltpu.SemaphoreType.DMA,
        ],
    )(_make_body(n_rows, seq_len, 1e-12))
    out = fn(idx2d, word_table, pos, gamma, beta)
    return out.reshape(b, seq_len, _HID)


# speedup vs baseline: 6.7983x; 1.0671x over previous
import functools

import jax
import jax.numpy as jnp
from jax import lax
from jax.experimental import pallas as pl
from jax.experimental.pallas import tpu as pltpu
from jax.experimental.pallas import tpu_sc as plsc

_HID = 64
_NREG = _HID // 16
_NC, _NS = 2, 16
_NW = _NC * _NS
_C = 512
_IG = 128
_G = _C // _IG
_NBUF = 3


def _rsqrt16(x):
    i = lax.bitcast_convert_type(x, jnp.int32)
    y = lax.bitcast_convert_type(jnp.int32(0x5F3759DF) - (i >> 1), jnp.float32)
    hx = x * 0.5
    for _ in range(2):
        y = y * (1.5 - hx * y * y)
    return y


def _make_body(n_rows, seq_len, eps):
    rows_per_w = n_rows // _NW
    n_chunks = rows_per_w // _C
    n_trips = n_chunks // _NBUF
    n_tail = n_chunks - n_trips * _NBUF
    if n_tail < 2:
        n_trips -= 1
        n_tail += _NBUF
    assert n_trips >= 1 and n_tail >= 2

    def body(idx_hbm, word_hbm, pos_hbm, gamma_hbm, beta_hbm, out_hbm,
             idx0, idx1, idx2, rows0, rows1, rows2, pos_v, gamma_v, beta_v,
             gsem0, gsem1, gsem2, ssem0, ssem1, ssem2, isem0, isem1, isem2):
        idx_b = (idx0, idx1, idx2)
        rows_b = (rows0, rows1, rows2)
        gsem_b = (gsem0, gsem1, gsem2)
        ssem_b = (ssem0, ssem1, ssem2)
        isem_b = (isem0, isem1, isem2)

        wid = lax.axis_index("s") * _NC + lax.axis_index("c")
        base = wid * rows_per_w
        pltpu.sync_copy(pos_hbm, pos_v)
        pltpu.sync_copy(gamma_hbm, gamma_v)
        pltpu.sync_copy(beta_hbm, beta_v)
        g_regs = [gamma_v[pl.ds(16 * k, 16)] for k in range(_NREG)]
        b_regs = [beta_v[pl.ds(16 * k, 16)] for k in range(_NREG)]
        lanes = lax.iota(jnp.int32, 16)
        perms = [lanes ^ d for d in (1, 2, 4, 8)]

        def fire_idx(slot, g):
            cbase = base + g * _C
            pltpu.async_copy(idx_hbm.at[pl.ds(lax.div(cbase, _IG), _G)],
                             idx_b[slot], isem_b[slot])

        def wait_idx(slot):
            pltpu.make_async_copy(idx_hbm.at[pl.ds(0, _G)], idx_b[slot],
                                  isem_b[slot]).wait()

        def fire_gather(buf, g):
            for j in range(_G):
                pltpu.async_copy(word_hbm.at[idx_b[buf].at[j]],
                                 rows_b[buf].at[pl.ds(j * _IG, _IG)],
                                 gsem_b[buf])

        def wait_gather(buf):
            pltpu.make_async_copy(word_hbm.at[pl.ds(0, _C)], rows_b[buf],
                                  gsem_b[buf]).wait()

        def fire_store(buf, g):
            cbase = base + g * _C
            pltpu.async_copy(rows_b[buf], out_hbm.at[pl.ds(cbase, _C)],
                             ssem_b[buf])

        def wait_store(buf):
            pltpu.make_async_copy(rows_b[buf], out_hbm.at[pl.ds(0, _C)],
                                  ssem_b[buf]).wait()

        def compute(buf, cbase):
            rows_v = rows_b[buf]
            off = lax.rem(cbase, seq_len)

            @plsc.parallel_loop(0, _C, unroll=4)
            def row_body(r):
                p = r + off
                for _ in range((_C + seq_len - 1) // seq_len + 1):
                    p = jnp.where(p >= seq_len, p - seq_len, p)
                s = [rows_v[r, pl.ds(16 * k, 16)]
                     + pos_v[p, pl.ds(16 * k, 16)] for k in range(_NREG)]
                t = (s[0] + s[1]) + (s[2] + s[3])
                u = (s[0] * s[0] + s[1] * s[1]) + (s[2] * s[2] + s[3] * s[3])
                for perm in perms:
                    t = t + t.at[perm].get(mode="promise_in_bounds")
                    u = u + u.at[perm].get(mode="promise_in_bounds")
                m = t * (1.0 / _HID)
                var = jnp.maximum(u * (1.0 / _HID) - m * m, 0.0) + eps
                rs = _rsqrt16(var)
                for k in range(_NREG):
                    y = (s[k] - m) * rs
                    rows_v[r, pl.ds(16 * k, 16)] = y * g_regs[k] + b_regs[k]

        fire_idx(0, 0)
        wait_idx(0)
        fire_gather(0, 0)
        fire_idx(1, 1)

        def trip_body(tt, carry):
            c0 = _NBUF * tt
            for par in range(_NBUF):
                nb = (par + 1) % _NBUF
                ns = (par + 2) % _NBUF
                @pl.when(c0 + par >= 2)
                def _():
                    wait_store(nb)
                wait_idx(nb)
                fire_gather(nb, c0 + par + 1)
                fire_idx(ns, c0 + par + 2)
                wait_gather(par)
                compute(par, base + (c0 + par) * _C)
                fire_store(par, c0 + par)
            return carry

        lax.fori_loop(0, n_trips, trip_body, 0)

        for c in range(n_trips * _NBUF, n_chunks):
            buf = c % _NBUF
            nb = (c + 1) % _NBUF
            if c + 1 < n_chunks:
                wait_store(nb)
                wait_idx(nb)
                fire_gather(nb, c + 1)
            if c + 2 < n_chunks:
                fire_idx((c + 2) % _NBUF, c + 2)
            wait_gather(buf)
            compute(buf, base + c * _C)
            fire_store(buf, c)

        for c in range(n_chunks - _NBUF, n_chunks):
            wait_store(c % _NBUF)

    return body


def kernel(input_id, word_table, pos_table, gamma, beta):
    b, seq_len = input_id.shape
    n_rows = b * seq_len
    idx2d = input_id.reshape(n_rows // _IG, _IG).astype(jnp.int32)
    pos = pos_table[:seq_len]
    mesh = plsc.VectorSubcoreMesh(core_axis_name="c", subcore_axis_name="s")
    fn = functools.partial(
        pl.kernel,
        mesh=mesh,
        compiler_params=pltpu.CompilerParams(use_tc_tiling_on_sc=False),
        out_type=jax.ShapeDtypeStruct((n_rows, _HID), jnp.float32),
        scratch_types=(
            [pltpu.VMEM((_G, _IG), jnp.int32)] * _NBUF
            + [pltpu.VMEM((_C, _HID), jnp.float32)] * _NBUF
            + [pltpu.VMEM((seq_len, _HID), jnp.float32),
               pltpu.VMEM((_HID,), jnp.float32),
               pltpu.VMEM((_HID,), jnp.float32)]
            + [pltpu.SemaphoreType.DMA] * (3 * _NBUF)
        ),
    )(_make_body(n_rows, seq_len, 1e-12))
    out = fn(idx2d, word_table, pos, gamma, beta)
    return out.reshape(b, seq_len, _HID)


# speedup vs baseline: 6.8111x; 1.0019x over previous
import functools

import jax
import jax.numpy as jnp
from jax import lax
from jax.experimental import pallas as pl
from jax.experimental.pallas import tpu as pltpu
from jax.experimental.pallas import tpu_sc as plsc

_HID = 64
_NREG = _HID // 16
_NC, _NS = 2, 16
_NW = _NC * _NS
_C = 400
_IG = 80
_G = _C // _IG
_NBUF = 3


def _rsqrt16(x):
    i = lax.bitcast_convert_type(x, jnp.int32)
    y = lax.bitcast_convert_type(jnp.int32(0x5F3759DF) - (i >> 1), jnp.float32)
    hx = x * 0.5
    for _ in range(2):
        y = y * (1.5 - hx * y * y)
    return y


def _make_body(n_rows, seq_len, eps):
    rows_per_w = n_rows // _NW
    n_chunks = rows_per_w // _C
    seq_per_chunk = _C // seq_len
    assert _C % seq_len == 0 and rows_per_w % _C == 0
    n_trips = n_chunks // _NBUF
    n_tail = n_chunks - n_trips * _NBUF
    if n_tail < 2:
        n_trips -= 1
        n_tail += _NBUF
    assert n_trips >= 1 and n_tail >= 2

    def body(idx_hbm, word_hbm, pos_hbm, gamma_hbm, beta_hbm, out_hbm,
             idx0, idx1, idx2, rows0, rows1, rows2, pos_v, gamma_v, beta_v,
             gsem0, gsem1, gsem2, ssem0, ssem1, ssem2, isem0, isem1, isem2):
        idx_b = (idx0, idx1, idx2)
        rows_b = (rows0, rows1, rows2)
        gsem_b = (gsem0, gsem1, gsem2)
        ssem_b = (ssem0, ssem1, ssem2)
        isem_b = (isem0, isem1, isem2)

        wid = lax.axis_index("s") * _NC + lax.axis_index("c")
        base = wid * rows_per_w
        srow0 = wid * (rows_per_w // seq_len)
        pltpu.sync_copy(pos_hbm, pos_v)
        pltpu.sync_copy(gamma_hbm, gamma_v)
        pltpu.sync_copy(beta_hbm, beta_v)
        g_regs = [gamma_v[pl.ds(16 * k, 16)] for k in range(_NREG)]
        b_regs = [beta_v[pl.ds(16 * k, 16)] for k in range(_NREG)]
        lanes = lax.iota(jnp.int32, 16)
        perms = [lanes ^ d for d in (1, 2, 4, 8)]

        def fire_idx(slot, g):
            cbase = base + g * _C
            pltpu.async_copy(idx_hbm.at[pl.ds(lax.div(cbase, _IG), _G)],
                             idx_b[slot], isem_b[slot])

        def wait_idx(slot):
            pltpu.make_async_copy(idx_hbm.at[pl.ds(0, _G)], idx_b[slot],
                                  isem_b[slot]).wait()

        def fire_gather(buf, g):
            for j in range(_G):
                pltpu.async_copy(word_hbm.at[idx_b[buf].at[j]],
                                 rows_b[buf].at[pl.ds(j * _IG, _IG)],
                                 gsem_b[buf])

        def wait_gather(buf):
            pltpu.make_async_copy(word_hbm.at[pl.ds(0, _C)], rows_b[buf],
                                  gsem_b[buf]).wait()

        def fire_store(buf, g):
            srow = srow0 + g * seq_per_chunk
            for j in range(seq_per_chunk):
                pltpu.async_copy(rows_b[buf].at[pl.ds(j * seq_len, seq_len)],
                                 out_hbm.at[srow + j], ssem_b[buf])

        def wait_store(buf):
            for j in range(seq_per_chunk):
                pltpu.make_async_copy(
                    rows_b[buf].at[pl.ds(j * seq_len, seq_len)],
                    out_hbm.at[0], ssem_b[buf]).wait()

        def compute(buf, cbase):
            rows_v = rows_b[buf]

            @plsc.parallel_loop(0, _C, unroll=4)
            def row_body(r):
                p = r
                for _ in range(seq_per_chunk - 1):
                    p = jnp.where(p >= seq_len, p - seq_len, p)
                s = [rows_v[r, pl.ds(16 * k, 16)]
                     + pos_v[p, pl.ds(16 * k, 16)] for k in range(_NREG)]
                t = (s[0] + s[1]) + (s[2] + s[3])
                u = (s[0] * s[0] + s[1] * s[1]) + (s[2] * s[2] + s[3] * s[3])
                for perm in perms:
                    t = t + t.at[perm].get(mode="promise_in_bounds")
                    u = u + u.at[perm].get(mode="promise_in_bounds")
                m = t * (1.0 / _HID)
                var = jnp.maximum(u * (1.0 / _HID) - m * m, 0.0) + eps
                rs = _rsqrt16(var)
                for k in range(_NREG):
                    y = (s[k] - m) * rs
                    rows_v[r, pl.ds(16 * k, 16)] = y * g_regs[k] + b_regs[k]

        fire_idx(0, 0)
        wait_idx(0)
        fire_gather(0, 0)
        fire_idx(1, 1)

        def trip_body(tt, carry):
            c0 = _NBUF * tt
            for par in range(_NBUF):
                nb = (par + 1) % _NBUF
                ns = (par + 2) % _NBUF
                @pl.when(c0 + par >= 2)
                def _():
                    wait_store(nb)
                wait_idx(nb)
                fire_gather(nb, c0 + par + 1)
                fire_idx(ns, c0 + par + 2)
                wait_gather(par)
                compute(par, base + (c0 + par) * _C)
                fire_store(par, c0 + par)
            return carry

        lax.fori_loop(0, n_trips, trip_body, 0)

        for c in range(n_trips * _NBUF, n_chunks):
            buf = c % _NBUF
            nb = (c + 1) % _NBUF
            if c + 1 < n_chunks:
                wait_store(nb)
                wait_idx(nb)
                fire_gather(nb, c + 1)
            if c + 2 < n_chunks:
                fire_idx((c + 2) % _NBUF, c + 2)
            wait_gather(buf)
            compute(buf, base + c * _C)
            fire_store(buf, c)

        for c in range(n_chunks - _NBUF, n_chunks):
            wait_store(c % _NBUF)

    return body


def kernel(input_id, word_table, pos_table, gamma, beta):
    b, seq_len = input_id.shape
    n_rows = b * seq_len
    idx2d = input_id.reshape(n_rows // _IG, _IG).astype(jnp.int32)
    pos = pos_table[:seq_len]
    mesh = plsc.VectorSubcoreMesh(core_axis_name="c", subcore_axis_name="s")
    fn = functools.partial(
        pl.kernel,
        mesh=mesh,
        compiler_params=pltpu.CompilerParams(use_tc_tiling_on_sc=False),
        out_type=jax.ShapeDtypeStruct((b, seq_len, _HID), jnp.float32),
        scratch_types=(
            [pltpu.VMEM((_G, _IG), jnp.int32)] * _NBUF
            + [pltpu.VMEM((_C, _HID), jnp.float32)] * _NBUF
            + [pltpu.VMEM((seq_len, _HID), jnp.float32),
               pltpu.VMEM((_HID,), jnp.float32),
               pltpu.VMEM((_HID,), jnp.float32)]
            + [pltpu.SemaphoreType.DMA] * (3 * _NBUF)
        ),
    )(_make_body(n_rows, seq_len, 1e-12))
    return fn(idx2d, word_table, pos, gamma, beta)
